# baseline (speedup 1.0000x reference)
import jax
import jax.numpy as jnp
from jax.experimental import pallas as pl
from jax.experimental.pallas import tpu as pltpu

_EXPERTS = 64
_TOP_K = 8
_ALPHA = 0.01
_BLOCK = 1024
_CHUNK = 64
_NSTREAMS = 8
_SUB = _BLOCK // _NSTREAMS


def _router_kernel(x0, x1, x2, x3, x4, x5, x6, x7, w_ref, idx_ref,
                   wt_ref, aux_ref, pi_acc, cnt_acc):
    i = pl.program_id(0)
    nsteps = pl.num_programs(0)

    @pl.when(i == 0)
    def _init():
        pi_acc[...] = jnp.zeros_like(pi_acc)
        cnt_acc[...] = jnp.zeros_like(cnt_acc)

    w = w_ref[...]
    lane_f = jax.lax.broadcasted_iota(jnp.int32, (_CHUNK, _EXPERTS), 1).astype(
        jnp.float32
    )
    pi_part = None
    cnt_part = None
    for c in range(_BLOCK // _CHUNK):
        lo = c * _CHUNK
        x_ref = (x0, x1, x2, x3, x4, x5, x6, x7)[lo // _SUB]
        x = x_ref[lo % _SUB : lo % _SUB + _CHUNK, :]
        logits = jax.lax.dot_general(
            x, w, (((1,), (1,)), ((), ())), preferred_element_type=jnp.float32
        )
        e = jnp.exp(logits)
        rs = 1.0 / jnp.sum(e, axis=-1, keepdims=True)

        work = e
        idx_cols = []
        wt_cols = []
        for _ in range(_TOP_K):
            mx = jnp.max(work, axis=-1, keepdims=True)
            cand = jnp.where(work == mx, lane_f, 64.0)
            idxf = jnp.min(cand, axis=-1, keepdims=True)
            idx_cols.append(idxf)
            wt_cols.append(mx * rs)
            work = jnp.where(cand == idxf, 0.0, work)

        idx_ref[lo : lo + _CHUNK, :] = jnp.concatenate(
            idx_cols, axis=1
        ).astype(jnp.int32)
        wt_ref[lo : lo + _CHUNK, :] = jnp.concatenate(wt_cols, axis=1)

        chosen = (work != e).astype(jnp.float32)
        p = jnp.sum(e * rs, axis=0, keepdims=True)
        q = jnp.sum(chosen, axis=0, keepdims=True)
        pi_part = p if pi_part is None else pi_part + p
        cnt_part = q if cnt_part is None else cnt_part + q

    pi_acc[...] += pi_part
    cnt_acc[...] += cnt_part

    @pl.when(i == nsteps - 1)
    def _finalize():
        n_tokens = nsteps * _BLOCK
        scale = _EXPERTS * _ALPHA / (float(n_tokens) * float(n_tokens) * _TOP_K)
        aux = jnp.sum(pi_acc[...] * cnt_acc[...], keepdims=True) * scale
        aux_ref[...] = aux.reshape(1, 1)


def kernel(hidden_states, weight):
    b, s, h = hidden_states.shape
    n = b * s
    hs = hidden_states.reshape(n, h)
    grid = (n // _BLOCK,)
    ns = _NSTREAMS

    def mk_spec(j):
        return pl.BlockSpec((_SUB, h), lambda i, j=j: (i * ns + j, 0))

    idx, wt, aux = pl.pallas_call(
        _router_kernel,
        grid=grid,
        in_specs=[mk_spec(j) for j in range(ns)]
        + [pl.BlockSpec((_EXPERTS, h), lambda i: (0, 0))],
        out_specs=[
            pl.BlockSpec((_BLOCK, _TOP_K), lambda i: (i, 0)),
            pl.BlockSpec((_BLOCK, _TOP_K), lambda i: (i, 0)),
            pl.BlockSpec((1, 1), lambda i: (0, 0)),
        ],
        out_shape=[
            jax.ShapeDtypeStruct((n, _TOP_K), jnp.int32),
            jax.ShapeDtypeStruct((n, _TOP_K), jnp.float32),
            jax.ShapeDtypeStruct((1, 1), jnp.float32),
        ],
        scratch_shapes=[
            pltpu.VMEM((1, _EXPERTS), jnp.float32),
            pltpu.VMEM((1, _EXPERTS), jnp.float32),
        ],
    )(*([hs] * ns), weight)
    return idx, wt, aux[0, 0]
